# baseline (speedup 1.0000x reference)
import functools

import jax
import jax.numpy as jnp
from jax import lax
from jax.experimental import pallas as pl
from jax.experimental.pallas import tpu as pltpu
from jax.experimental.pallas import tpu_sc as plsc

NC = 2
NS = 16
L = 16

B = 16384
D = 64
SD = 72
N = 1000000
NW = NC * NS
BPW = B // NW
GROUPS = BPW // L

W = 128
NBLK = (N + W - 1) // W
TAIL_BLK = N // W
TAIL_W = N - TAIL_BLK * W


def _scan_tables_sc(uT, iT, ubT, ibT, tailu, taili, tailub, tailib,
                    uid, iid):
    mesh = plsc.VectorSubcoreMesh(core_axis_name="c", subcore_axis_name="s")

    @functools.partial(
        pl.kernel,
        out_type=(
            jax.ShapeDtypeStruct((B * SD,), jnp.float32),
            jax.ShapeDtypeStruct((B * SD,), jnp.float32),
        ),
        mesh=mesh,
        compiler_params=pltpu.CompilerParams(needs_layout_passes=False),
        scratch_types=[
            pltpu.VMEM((B + L,), jnp.int32),
            pltpu.VMEM((B + L,), jnp.int32),
            pltpu.VMEM((D, W), jnp.float32),
            pltpu.VMEM((D, W), jnp.float32),
            pltpu.VMEM((D, W), jnp.float32),
            pltpu.VMEM((D, W), jnp.float32),
            pltpu.VMEM((D, W), jnp.float32),
            pltpu.VMEM((D, W), jnp.float32),
            pltpu.VMEM((D, W), jnp.float32),
            pltpu.VMEM((D, W), jnp.float32),
            pltpu.VMEM((D, W), jnp.float32),
            pltpu.VMEM((D, W), jnp.float32),
            pltpu.VMEM((1, W + L), jnp.float32),
            pltpu.VMEM((1, W + L), jnp.float32),
            pltpu.VMEM((1, W + L), jnp.float32),
            pltpu.VMEM((1, W + L), jnp.float32),
            pltpu.VMEM((1, W + L), jnp.float32),
            pltpu.VMEM((1, W + L), jnp.float32),
            pltpu.VMEM((1, W + L), jnp.float32),
            pltpu.VMEM((1, W + L), jnp.float32),
            pltpu.VMEM((1, W + L), jnp.float32),
            pltpu.VMEM((1, W + L), jnp.float32),
            pltpu.VMEM((TAIL_W * D,), jnp.float32),
            pltpu.VMEM((TAIL_W + L,), jnp.float32),
            pltpu.VMEM((16 * SD,), jnp.float32),
            pltpu.SMEM((2,), jnp.int32),
            pltpu.SMEM((256,), jnp.int32),
            pltpu.SemaphoreType.DMA,
            pltpu.SemaphoreType.DMA,
            pltpu.SemaphoreType.DMA,
            pltpu.SemaphoreType.DMA,
            pltpu.SemaphoreType.DMA,
            pltpu.SemaphoreType.DMA,
            pltpu.SemaphoreType.DMA,
            pltpu.SemaphoreType.DMA,
            pltpu.SemaphoreType.DMA,
            pltpu.SemaphoreType.DMA,
            pltpu.SemaphoreType.DMA,
        ],
    )
    def ka(uT_hbm, iT_hbm, ubT_hbm, ibT_hbm, tailu_hbm, taili_hbm,
           tailub_hbm, tailib_hbm, uid_hbm, iid_hbm,
           su_hbm, si_hbm,
           ids_v, hits_v, blk0, blk1, blk2, blk3,
           blk4, blk5, blk6, blk7, blk8, blk9,
           bb0, bb1, bb2, bb3, bb4, bb5, bb6, bb7, bb8, bb9,
           tailv, tailbv, rowbuf,
           rst, smc, f0, f1, f2, f3, f4, f5, f6, f7, f8, f9, rsem):
        hits2_v = ids_v
        wid = lax.axis_index("s") * NC + lax.axis_index("c")
        lo = (NBLK * wid) // NW
        hi = (NBLK * (wid + 1)) // NW
        lo_col = lo * W
        hi_col = hi * W
        hi_main = jnp.minimum(hi, TAIL_BLK)
        lanes = lax.iota(jnp.int32, L)

        rst[0] = 0
        rst[1] = 0

        def do_table(tbl_hbm, bias_hbm, tail_hbm, tailb_hbm, ids_hbm,
                     staged_hbm):
            def fetch(gc, buf, bbuf, sem):
                off = pl.multiple_of(gc * W, 128)
                cp = pltpu.async_copy(
                    tbl_hbm.at[:, pl.ds(off, W)], buf, sem)
                pltpu.async_copy(
                    bias_hbm.at[:, pl.ds(off, W)],
                    bbuf.at[:, pl.ds(0, W)], sem)
                return cp

            bufs = ((blk0, bb0, f0), (blk1, bb1, f1),
                    (blk2, bb2, f2), (blk3, bb3, f3),
                    (blk4, bb4, f4), (blk5, bb5, f5),
                    (blk6, bb6, f6), (blk7, bb7, f7),
                    (blk8, bb8, f8), (blk9, bb9, f9))
            for i, (buf, bbuf, sem) in enumerate(bufs):
                @pl.when(lo + i < hi_main)
                def _(buf=buf, bbuf=bbuf, sem=sem, i=i):
                    fetch(lo + i, buf, bbuf, sem)

            pltpu.sync_copy(ids_hbm, ids_v.at[pl.ds(0, B)])

            def zero(i, _):
                smc[i] = 0
                return 0

            lax.fori_loop(0, 256, zero, 0)

            def scan(j, cursor):
                v = ids_v[pl.ds(j * L, L)]
                m = (v >= lo_col) & (v < hi_col)
                packed = (v - lo_col) * B + (j * L + lanes)
                plsc.store_compressed(
                    hits_v.at[pl.ds(cursor, L)], packed, mask=m)
                return cursor + plsc.all_reduce_population_count(m)[0]

            nh = lax.fori_loop(0, B // L, scan, jnp.int32(0))

            def hist(q, _):
                h = hits_v[pl.ds(q, L)][0]
                blkq = h >> 21
                smc[blkq] = smc[blkq] + 1
                return 0

            lax.fori_loop(0, nh, hist, 0)

            def pfx(i, run):
                c = smc[i]
                smc[i] = run
                return run + c

            lax.fori_loop(0, 256, pfx, jnp.int32(0))

            def place(q, _):
                h = hits_v[pl.ds(q, L)][0]
                blkq = h >> 21
                pos = smc[blkq]
                smc[blkq] = pos + 1
                plsc.store_scatter(
                    hits2_v,
                    [jnp.zeros((L,), jnp.int32) + pos],
                    jnp.zeros((L,), jnp.int32) + h,
                    mask=lanes == 0)
                return 0

            lax.fori_loop(0, nh, place, 0)

            def wait_one_row():
                pltpu.make_async_copy(
                    rowbuf.at[pl.ds(0, SD)],
                    staged_hbm.at[pl.ds(0, SD)], rsem).wait()

            def emit_row(buf, bbuf, from_tail, col1, b1):
                slot = rst[0]
                infl = rst[1]

                @pl.when(infl >= 16)
                def _():
                    wait_one_row()

                infl = jnp.where(infl >= 16, infl - 1, infl)
                if from_tail:
                    for kk in range(4):
                        v = buf[pl.ds(col1 * D + kk * L, L)]
                        rowbuf[pl.ds(slot * SD + kk * L, L)] = v
                    bv = bbuf[pl.ds(col1, L)][0]
                else:
                    colv = jnp.zeros((L,), jnp.int32) + col1
                    for kk in range(4):
                        v = plsc.load_gather(buf, [lanes + kk * L, colv])
                        rowbuf[pl.ds(slot * SD + kk * L, L)] = v
                    bv = bbuf.at[0][pl.ds(col1, L)][0]
                plsc.store_scatter(
                    rowbuf,
                    [jnp.zeros((L,), jnp.int32) + (slot * SD + D)],
                    jnp.zeros((L,), jnp.float32) + bv,
                    mask=lanes == 0)
                pltpu.async_copy(
                    rowbuf.at[pl.ds(slot * SD, SD)],
                    staged_hbm.at[pl.ds(b1 * SD, SD)], rsem)
                rst[0] = (slot + 1) & 15
                rst[1] = infl + 1

            def process(buf, bbuf, t, from_tail):
                s = jnp.where(t == 0, 0, smc[jnp.maximum(t - 1, 0)])
                e = smc[t]

                def hit(q, _):
                    h1 = hits2_v[pl.ds(q, L)][0]
                    b1 = h1 & (B - 1)
                    col1 = (h1 >> 14) - t * W
                    emit_row(buf, bbuf, from_tail, col1, b1)
                    return 0

                lax.fori_loop(s, e, hit, 0)

            def drain_all():
                def w(i, _):
                    wait_one_row()
                    return 0

                lax.fori_loop(0, rst[1], w, 0)
                rst[1] = 0

            def wait_fetch(buf, bbuf, sem):
                pltpu.make_async_copy(
                    tbl_hbm.at[:, pl.ds(0, W)], buf, sem).wait()
                pltpu.make_async_copy(
                    bias_hbm.at[:, pl.ds(0, W)],
                    bbuf.at[:, pl.ds(0, W)], sem).wait()

            nquads = (hi_main - lo + 9) // 10

            def quad(p, _):
                cbase = lo + 10 * p
                for i, (buf, bbuf, sem) in enumerate(bufs):
                    c = cbase + i

                    @pl.when(c < hi_main)
                    def _(buf=buf, bbuf=bbuf, sem=sem, c=c):
                        wait_fetch(buf, bbuf, sem)
                        process(buf, bbuf, c - lo, False)

                        @pl.when(c + 10 < hi_main)
                        def _():
                            fetch(c + 10, buf, bbuf, sem)

                return 0

            lax.fori_loop(0, nquads, quad, 0)

            @pl.when(hi == NBLK)
            def _():
                pltpu.sync_copy(tail_hbm, tailv)
                pltpu.sync_copy(tailb_hbm, tailbv.at[pl.ds(0, TAIL_W)])
                process(tailv, tailbv, TAIL_BLK - lo, True)

            drain_all()

        do_table(uT_hbm, ubT_hbm, tailu_hbm, tailub_hbm, uid_hbm, su_hbm)
        do_table(iT_hbm, ibT_hbm, taili_hbm, tailib_hbm, iid_hbm, si_hbm)

    return ka(uT, iT, ubT, ibT, tailu, taili, tailub, tailib, uid, iid)


def _dot_sc(su, si):
    mesh = plsc.VectorSubcoreMesh(core_axis_name="c", subcore_axis_name="s")

    @functools.partial(
        pl.kernel,
        out_type=jax.ShapeDtypeStruct((B,), jnp.float32),
        mesh=mesh,
        compiler_params=pltpu.CompilerParams(
            use_tc_tiling_on_sc=False, needs_layout_passes=False),
        scratch_types=[
            pltpu.VMEM((BPW * SD,), jnp.float32),
            pltpu.VMEM((BPW * SD,), jnp.float32),
            pltpu.VMEM((BPW,), jnp.float32),
            pltpu.SemaphoreType.DMA,
            pltpu.SemaphoreType.DMA,
        ],
    )
    def kb(su_hbm, si_hbm, out_hbm, urows_v, irows_v, outv, sem0, sem1):
        wid = lax.axis_index("s") * NC + lax.axis_index("c")
        base = wid * BPW

        cp0 = pltpu.async_copy(
            su_hbm.at[pl.ds(base * SD, BPW * SD)], urows_v, sem0)
        cp1 = pltpu.async_copy(
            si_hbm.at[pl.ds(base * SD, BPW * SD)], irows_v, sem1)
        cp0.wait()
        cp1.wait()

        lanes = lax.iota(jnp.int32, L)

        def group(g, _):
            rows = lanes + g * L
            flatb = rows * SD + D
            acc = (plsc.load_gather(urows_v, [flatb])
                   + plsc.load_gather(irows_v, [flatb]))

            def dstep(d4, a):
                for u in range(4):
                    flat = rows * SD + (d4 * 4 + u)
                    uv = plsc.load_gather(urows_v, [flat])
                    iv = plsc.load_gather(irows_v, [flat])
                    a = a + uv * iv
                return a

            acc = lax.fori_loop(0, D // 4, dstep, acc)
            outv[pl.ds(g * L, L)] = acc
            return 0

        lax.fori_loop(0, GROUPS, group, 0)
        pltpu.sync_copy(outv, out_hbm.at[pl.ds(base, BPW)])

    return kb(su, si)


def kernel(x, u_emb, i_emb, u_bias, i_bias):
    uid = x[:, 0].astype(jnp.int32)
    iid = x[:, 1].astype(jnp.int32)
    tailu = u_emb[TAIL_BLK * W:].reshape(-1)
    taili = i_emb[TAIL_BLK * W:].reshape(-1)
    tailub = u_bias[TAIL_BLK * W:].reshape(-1)
    tailib = i_bias[TAIL_BLK * W:].reshape(-1)
    su, si = _scan_tables_sc(u_emb.T, i_emb.T, u_bias.T, i_bias.T,
                             tailu, taili, tailub, tailib, uid, iid)
    out = _dot_sc(su, si)
    return out.reshape(-1, 1)


# speedup vs baseline: 1.0235x; 1.0235x over previous
import functools

import jax
import jax.numpy as jnp
from jax import lax
from jax.experimental import pallas as pl
from jax.experimental.pallas import tpu as pltpu
from jax.experimental.pallas import tpu_sc as plsc

NC = 2
NS = 16
L = 16

B = 16384
D = 64
SD = 72
N = 1000000
NW = NC * NS
BPW = B // NW
GROUPS = BPW // L

W = 128
NBLK = (N + W - 1) // W
TAIL_BLK = N // W
TAIL_W = N - TAIL_BLK * W


def _scan_tables_sc(uT, iT, ubT, ibT, tailu, taili, tailub, tailib,
                    uid, iid):
    mesh = plsc.VectorSubcoreMesh(core_axis_name="c", subcore_axis_name="s")

    @functools.partial(
        pl.kernel,
        out_type=(
            jax.ShapeDtypeStruct((B * SD,), jnp.float32),
            jax.ShapeDtypeStruct((B * SD,), jnp.float32),
        ),
        mesh=mesh,
        compiler_params=pltpu.CompilerParams(needs_layout_passes=False),
        scratch_types=[
            pltpu.VMEM((B,), jnp.int32),
            pltpu.VMEM((B + L,), jnp.int32),
            pltpu.VMEM((B + L,), jnp.int32),
            pltpu.VMEM((D, W), jnp.float32),
            pltpu.VMEM((D, W), jnp.float32),
            pltpu.VMEM((D, W), jnp.float32),
            pltpu.VMEM((D, W), jnp.float32),
            pltpu.VMEM((D, W), jnp.float32),
            pltpu.VMEM((D, W), jnp.float32),
            pltpu.VMEM((D, W), jnp.float32),
            pltpu.VMEM((D, W), jnp.float32),
            pltpu.VMEM((1, W + L), jnp.float32),
            pltpu.VMEM((1, W + L), jnp.float32),
            pltpu.VMEM((1, W + L), jnp.float32),
            pltpu.VMEM((1, W + L), jnp.float32),
            pltpu.VMEM((1, W + L), jnp.float32),
            pltpu.VMEM((1, W + L), jnp.float32),
            pltpu.VMEM((1, W + L), jnp.float32),
            pltpu.VMEM((1, W + L), jnp.float32),
            pltpu.VMEM((TAIL_W * D,), jnp.float32),
            pltpu.VMEM((TAIL_W + L,), jnp.float32),
            pltpu.VMEM((16 * SD,), jnp.float32),
            pltpu.SMEM((2,), jnp.int32),
            pltpu.SMEM((256,), jnp.int32),
            pltpu.SemaphoreType.DMA,
            pltpu.SemaphoreType.DMA,
            pltpu.SemaphoreType.DMA,
            pltpu.SemaphoreType.DMA,
            pltpu.SemaphoreType.DMA,
            pltpu.SemaphoreType.DMA,
            pltpu.SemaphoreType.DMA,
            pltpu.SemaphoreType.DMA,
            pltpu.SemaphoreType.DMA,
        ],
    )
    def ka(uT_hbm, iT_hbm, ubT_hbm, ibT_hbm, tailu_hbm, taili_hbm,
           tailub_hbm, tailib_hbm, uid_hbm, iid_hbm,
           su_hbm, si_hbm,
           ids_v, hits_v, hits2_v, blk0, blk1, blk2, blk3,
           blk4, blk5, blk6, blk7,
           bb0, bb1, bb2, bb3, bb4, bb5, bb6, bb7, tailv, tailbv, rowbuf,
           rst, smc, f0, f1, f2, f3, f4, f5, f6, f7, rsem):
        wid = lax.axis_index("s") * NC + lax.axis_index("c")
        lo = (NBLK * wid) // NW
        hi = (NBLK * (wid + 1)) // NW
        lo_col = lo * W
        hi_col = hi * W
        hi_main = jnp.minimum(hi, TAIL_BLK)
        lanes = lax.iota(jnp.int32, L)

        rst[0] = 0
        rst[1] = 0

        def do_table(tbl_hbm, bias_hbm, tail_hbm, tailb_hbm, ids_hbm,
                     staged_hbm):
            def fetch(gc, buf, bbuf, sem):
                off = pl.multiple_of(gc * W, 128)
                cp = pltpu.async_copy(
                    tbl_hbm.at[:, pl.ds(off, W)], buf, sem)
                pltpu.async_copy(
                    bias_hbm.at[:, pl.ds(off, W)],
                    bbuf.at[:, pl.ds(0, W)], sem)
                return cp

            bufs = ((blk0, bb0, f0), (blk1, bb1, f1),
                    (blk2, bb2, f2), (blk3, bb3, f3),
                    (blk4, bb4, f4), (blk5, bb5, f5),
                    (blk6, bb6, f6), (blk7, bb7, f7))
            for i, (buf, bbuf, sem) in enumerate(bufs):
                @pl.when(lo + i < hi_main)
                def _(buf=buf, bbuf=bbuf, sem=sem, i=i):
                    fetch(lo + i, buf, bbuf, sem)

            pltpu.sync_copy(ids_hbm, ids_v)

            def zero(i, _):
                smc[i] = 0
                return 0

            lax.fori_loop(0, 256, zero, 0)

            def scan(j, cursor):
                v = ids_v[pl.ds(j * L, L)]
                m = (v >= lo_col) & (v < hi_col)
                packed = (v - lo_col) * B + (j * L + lanes)
                plsc.store_compressed(
                    hits_v.at[pl.ds(cursor, L)], packed, mask=m)
                return cursor + plsc.all_reduce_population_count(m)[0]

            nh = lax.fori_loop(0, B // L, scan, jnp.int32(0))

            def hist(q, _):
                h = hits_v[pl.ds(q, L)][0]
                blkq = h >> 21
                smc[blkq] = smc[blkq] + 1
                return 0

            lax.fori_loop(0, nh, hist, 0)

            def pfx(i, run):
                c = smc[i]
                smc[i] = run
                return run + c

            lax.fori_loop(0, 256, pfx, jnp.int32(0))

            def place(q, _):
                h = hits_v[pl.ds(q, L)][0]
                blkq = h >> 21
                pos = smc[blkq]
                smc[blkq] = pos + 1
                plsc.store_scatter(
                    hits2_v,
                    [jnp.zeros((L,), jnp.int32) + pos],
                    jnp.zeros((L,), jnp.int32) + h,
                    mask=lanes == 0)
                return 0

            lax.fori_loop(0, nh, place, 0)

            def wait_one_row():
                pltpu.make_async_copy(
                    rowbuf.at[pl.ds(0, SD)],
                    staged_hbm.at[pl.ds(0, SD)], rsem).wait()

            def emit_row(buf, bbuf, from_tail, col1, b1):
                slot = rst[0]
                infl = rst[1]

                @pl.when(infl >= 16)
                def _():
                    wait_one_row()

                infl = jnp.where(infl >= 16, infl - 1, infl)
                if from_tail:
                    for kk in range(4):
                        v = buf[pl.ds(col1 * D + kk * L, L)]
                        rowbuf[pl.ds(slot * SD + kk * L, L)] = v
                    bv = bbuf[pl.ds(col1, L)][0]
                else:
                    colv = jnp.zeros((L,), jnp.int32) + col1
                    for kk in range(4):
                        v = plsc.load_gather(buf, [lanes + kk * L, colv])
                        rowbuf[pl.ds(slot * SD + kk * L, L)] = v
                    bv = bbuf.at[0][pl.ds(col1, L)][0]
                plsc.store_scatter(
                    rowbuf,
                    [jnp.zeros((L,), jnp.int32) + (slot * SD + D)],
                    jnp.zeros((L,), jnp.float32) + bv,
                    mask=lanes == 0)
                pltpu.async_copy(
                    rowbuf.at[pl.ds(slot * SD, SD)],
                    staged_hbm.at[pl.ds(b1 * SD, SD)], rsem)
                rst[0] = (slot + 1) & 15
                rst[1] = infl + 1

            def process(buf, bbuf, t, from_tail):
                s = jnp.where(t == 0, 0, smc[jnp.maximum(t - 1, 0)])
                e = smc[t]

                def hit(q, _):
                    h1 = hits2_v[pl.ds(q, L)][0]
                    b1 = h1 & (B - 1)
                    col1 = (h1 >> 14) - t * W
                    emit_row(buf, bbuf, from_tail, col1, b1)
                    return 0

                lax.fori_loop(s, e, hit, 0)

            def drain_all():
                def w(i, _):
                    wait_one_row()
                    return 0

                lax.fori_loop(0, rst[1], w, 0)
                rst[1] = 0

            def wait_fetch(buf, bbuf, sem):
                pltpu.make_async_copy(
                    tbl_hbm.at[:, pl.ds(0, W)], buf, sem).wait()
                pltpu.make_async_copy(
                    bias_hbm.at[:, pl.ds(0, W)],
                    bbuf.at[:, pl.ds(0, W)], sem).wait()

            nquads = (hi_main - lo + 7) // 8

            def quad(p, _):
                cbase = lo + 8 * p
                for i, (buf, bbuf, sem) in enumerate(bufs):
                    c = cbase + i

                    @pl.when(c < hi_main)
                    def _(buf=buf, bbuf=bbuf, sem=sem, c=c):
                        wait_fetch(buf, bbuf, sem)
                        process(buf, bbuf, c - lo, False)

                        @pl.when(c + 8 < hi_main)
                        def _():
                            fetch(c + 8, buf, bbuf, sem)

                return 0

            lax.fori_loop(0, nquads, quad, 0)

            @pl.when(hi == NBLK)
            def _():
                pltpu.sync_copy(tail_hbm, tailv)
                pltpu.sync_copy(tailb_hbm, tailbv.at[pl.ds(0, TAIL_W)])
                process(tailv, tailbv, TAIL_BLK - lo, True)

            drain_all()

        do_table(uT_hbm, ubT_hbm, tailu_hbm, tailub_hbm, uid_hbm, su_hbm)
        do_table(iT_hbm, ibT_hbm, taili_hbm, tailib_hbm, iid_hbm, si_hbm)

    return ka(uT, iT, ubT, ibT, tailu, taili, tailub, tailib, uid, iid)


def _dot_sc(su, si):
    mesh = plsc.VectorSubcoreMesh(core_axis_name="c", subcore_axis_name="s")

    @functools.partial(
        pl.kernel,
        out_type=jax.ShapeDtypeStruct((B,), jnp.float32),
        mesh=mesh,
        compiler_params=pltpu.CompilerParams(
            use_tc_tiling_on_sc=False, needs_layout_passes=False),
        scratch_types=[
            pltpu.VMEM((BPW * SD,), jnp.float32),
            pltpu.VMEM((BPW * SD,), jnp.float32),
            pltpu.VMEM((BPW,), jnp.float32),
            pltpu.SemaphoreType.DMA,
            pltpu.SemaphoreType.DMA,
        ],
    )
    def kb(su_hbm, si_hbm, out_hbm, urows_v, irows_v, outv, sem0, sem1):
        wid = lax.axis_index("s") * NC + lax.axis_index("c")
        base = wid * BPW

        cp0 = pltpu.async_copy(
            su_hbm.at[pl.ds(base * SD, BPW * SD)], urows_v, sem0)
        cp1 = pltpu.async_copy(
            si_hbm.at[pl.ds(base * SD, BPW * SD)], irows_v, sem1)
        cp0.wait()
        cp1.wait()

        lanes = lax.iota(jnp.int32, L)

        def group(g, _):
            rows = lanes + g * L
            flatb = rows * SD + D
            acc = (plsc.load_gather(urows_v, [flatb])
                   + plsc.load_gather(irows_v, [flatb]))

            def dstep(d4, a):
                for u in range(4):
                    flat = rows * SD + (d4 * 4 + u)
                    uv = plsc.load_gather(urows_v, [flat])
                    iv = plsc.load_gather(irows_v, [flat])
                    a = a + uv * iv
                return a

            acc = lax.fori_loop(0, D // 4, dstep, acc)
            outv[pl.ds(g * L, L)] = acc
            return 0

        lax.fori_loop(0, GROUPS, group, 0)
        pltpu.sync_copy(outv, out_hbm.at[pl.ds(base, BPW)])

    return kb(su, si)


def kernel(x, u_emb, i_emb, u_bias, i_bias):
    uid = x[:, 0].astype(jnp.int32)
    iid = x[:, 1].astype(jnp.int32)
    tailu = u_emb[TAIL_BLK * W:].reshape(-1)
    taili = i_emb[TAIL_BLK * W:].reshape(-1)
    tailub = u_bias[TAIL_BLK * W:].reshape(-1)
    tailib = i_bias[TAIL_BLK * W:].reshape(-1)
    su, si = _scan_tables_sc(u_emb.T, i_emb.T, u_bias.T, i_bias.T,
                             tailu, taili, tailub, tailib, uid, iid)
    out = _dot_sc(su, si)
    return out.reshape(-1, 1)
